# Initial kernel scaffold; baseline (speedup 1.0000x reference)
import jax
import jax.numpy as jnp
from jax.experimental import pallas as pl


def kernel(hidden, weight, expert_bias):
    raise NotImplementedError("write your pallas kernel here")


# speedup vs baseline: 11.2299x; 11.2299x over previous
import jax
import jax.numpy as jnp
from jax.experimental import pallas as pl

HIDDEN = 2048
NUM_EXPERTS = 64
TOPK = 8
TOPK_SCALING_FACTOR = 2.5
BLK = 512


def _router_body(x_ref, wt_ref, b_ref, probs_ref, map_ref):
    x = x_ref[...]
    logits = jnp.dot(x, wt_ref[...], preferred_element_type=jnp.float32)
    sp = jnp.maximum(logits, 0.0) + jnp.log(1.0 + jnp.exp(-jnp.abs(logits)))
    scores = jnp.sqrt(sp)
    sel = scores + b_ref[...]
    iota = jax.lax.broadcasted_iota(jnp.int32, sel.shape, 1)
    mask = jnp.zeros(sel.shape, jnp.bool_)
    work = sel
    for _ in range(TOPK):
        m = jnp.max(work, axis=1, keepdims=True)
        idx = jnp.min(
            jnp.where(work == m, iota, NUM_EXPERTS), axis=1, keepdims=True
        )
        chosen = iota == idx
        mask = jnp.logical_or(mask, chosen)
        work = jnp.where(chosen, -jnp.inf, work)
    w = jnp.where(mask, scores, 0.0)
    denom = jnp.clip(jnp.sum(w, axis=1, keepdims=True), 1e-12, None)
    probs_ref[...] = jnp.where(mask, scores * (TOPK_SCALING_FACTOR / denom), 0.0)
    map_ref[...] = mask


def kernel(hidden, weight, expert_bias):
    flat = hidden.reshape(-1, HIDDEN)
    n = flat.shape[0]
    wt = weight.T
    bias = expert_bias.reshape(1, NUM_EXPERTS)
    probs, rmap = pl.pallas_call(
        _router_body,
        grid=(n // BLK,),
        in_specs=[
            pl.BlockSpec((BLK, HIDDEN), lambda i: (i, 0)),
            pl.BlockSpec((HIDDEN, NUM_EXPERTS), lambda i: (0, 0)),
            pl.BlockSpec((1, NUM_EXPERTS), lambda i: (0, 0)),
        ],
        out_specs=[
            pl.BlockSpec((BLK, NUM_EXPERTS), lambda i: (i, 0)),
            pl.BlockSpec((BLK, NUM_EXPERTS), lambda i: (i, 0)),
        ],
        out_shape=[
            jax.ShapeDtypeStruct((n, NUM_EXPERTS), jnp.float32),
            jax.ShapeDtypeStruct((n, NUM_EXPERTS), jnp.bool_),
        ],
    )(flat, wt, bias)
    return probs, rmap


# speedup vs baseline: 15.5815x; 1.3875x over previous
import jax
import jax.numpy as jnp
from jax.experimental import pallas as pl

HIDDEN = 2048
NUM_EXPERTS = 64
TOPK = 8
TOPK_SCALING_FACTOR = 2.5
BLK = 512


def _router_body(x_ref, wt_ref, b_ref, probs_ref, map_ref):
    x = x_ref[...]
    logits = jnp.dot(x, wt_ref[...], preferred_element_type=jnp.float32)
    sp = jnp.maximum(logits, 0.0) + jnp.log(1.0 + jnp.exp(-jnp.abs(logits)))
    scores = jnp.sqrt(sp)
    sel = scores + b_ref[...]
    iota = jax.lax.broadcasted_iota(jnp.int32, sel.shape, 1)
    mask = jnp.zeros(sel.shape, jnp.bool_)
    work = sel
    for _ in range(TOPK):
        idx = jnp.argmax(work, axis=1)[:, None]
        chosen = iota == idx
        mask = jnp.logical_or(mask, chosen)
        work = jnp.where(chosen, -jnp.inf, work)
    w = jnp.where(mask, scores, 0.0)
    denom = jnp.clip(jnp.sum(w, axis=1, keepdims=True), 1e-12, None)
    probs_ref[...] = jnp.where(mask, scores * (TOPK_SCALING_FACTOR / denom), 0.0)
    map_ref[...] = mask


def kernel(hidden, weight, expert_bias):
    flat = hidden.reshape(-1, HIDDEN)
    n = flat.shape[0]
    wt = weight.T
    bias = expert_bias.reshape(1, NUM_EXPERTS)
    probs, rmap = pl.pallas_call(
        _router_body,
        grid=(n // BLK,),
        in_specs=[
            pl.BlockSpec((BLK, HIDDEN), lambda i: (i, 0)),
            pl.BlockSpec((HIDDEN, NUM_EXPERTS), lambda i: (0, 0)),
            pl.BlockSpec((1, NUM_EXPERTS), lambda i: (0, 0)),
        ],
        out_specs=[
            pl.BlockSpec((BLK, NUM_EXPERTS), lambda i: (i, 0)),
            pl.BlockSpec((BLK, NUM_EXPERTS), lambda i: (i, 0)),
        ],
        out_shape=[
            jax.ShapeDtypeStruct((n, NUM_EXPERTS), jnp.float32),
            jax.ShapeDtypeStruct((n, NUM_EXPERTS), jnp.bool_),
        ],
    )(flat, wt, bias)
    return probs, rmap


# speedup vs baseline: 16.1151x; 1.0342x over previous
import jax
import jax.numpy as jnp
from jax.experimental import pallas as pl

HIDDEN = 2048
NUM_EXPERTS = 64
TOPK = 8
TOPK_SCALING_FACTOR = 2.5
BLK = 512


def _router_body(x_ref, wt_ref, b_ref, probs_ref, map_ref):
    x = x_ref[...]
    logits = jax.lax.dot_general(
        x, wt_ref[...], (((1,), (1,)), ((), ())),
        preferred_element_type=jnp.float32,
    )
    sp = jnp.maximum(logits, 0.0) + jnp.log(1.0 + jnp.exp(-jnp.abs(logits)))
    scores = jnp.sqrt(sp)
    sel = scores + b_ref[...]
    iota = jax.lax.broadcasted_iota(jnp.int32, sel.shape, 1)
    mask = jnp.zeros(sel.shape, jnp.bool_)
    work = sel
    for _ in range(TOPK):
        idx = jnp.argmax(work, axis=1)[:, None]
        chosen = iota == idx
        mask = jnp.logical_or(mask, chosen)
        work = jnp.where(chosen, -jnp.inf, work)
    w = jnp.where(mask, scores, 0.0)
    denom = jnp.clip(jnp.sum(w, axis=1, keepdims=True), 1e-12, None)
    probs_ref[...] = jnp.where(mask, scores * (TOPK_SCALING_FACTOR / denom), 0.0)
    map_ref[...] = mask


def kernel(hidden, weight, expert_bias):
    flat = hidden.reshape(-1, HIDDEN)
    n = flat.shape[0]
    bias = expert_bias.reshape(1, NUM_EXPERTS)
    probs, rmap = pl.pallas_call(
        _router_body,
        grid=(n // BLK,),
        in_specs=[
            pl.BlockSpec((BLK, HIDDEN), lambda i: (i, 0)),
            pl.BlockSpec((NUM_EXPERTS, HIDDEN), lambda i: (0, 0)),
            pl.BlockSpec((1, NUM_EXPERTS), lambda i: (0, 0)),
        ],
        out_specs=[
            pl.BlockSpec((BLK, NUM_EXPERTS), lambda i: (i, 0)),
            pl.BlockSpec((BLK, NUM_EXPERTS), lambda i: (i, 0)),
        ],
        out_shape=[
            jax.ShapeDtypeStruct((n, NUM_EXPERTS), jnp.float32),
            jax.ShapeDtypeStruct((n, NUM_EXPERTS), jnp.bool_),
        ],
    )(flat, weight, bias)
    return probs, rmap
